# Initial kernel scaffold; baseline (speedup 1.0000x reference)
import functools

import jax
import jax.numpy as jnp
from jax import lax
from jax.experimental import pallas as pl
from jax.experimental.pallas import tpu as pltpu
from jax.experimental.pallas import tpu_sc as plsc

HIDDEN = 128
DEPTH = 10
MAX_NEI = 8
NW = 32


@functools.partial(jax.jit, static_argnames=("chunk",))
def _sc_gather(table, idx, *, chunk):
    B = idx.shape[0]
    H = table.shape[1]
    b_per_w = B // NW
    n_chunks = b_per_w // chunk
    mesh = plsc.VectorSubcoreMesh(core_axis_name="c", subcore_axis_name="s")

    @functools.partial(
        pl.kernel,
        out_type=jax.ShapeDtypeStruct((B, H), jnp.float32),
        mesh=mesh,
        scratch_types=[
            pltpu.VMEM((chunk,), jnp.int32),
            pltpu.VMEM((chunk, H), jnp.float32),
            pltpu.SemaphoreType.DMA,
        ],
    )
    def gather_kernel(table_hbm, idx_hbm, out_hbm, idx_v, rows_v, sem):
        wid = lax.axis_index("s") * 2 + lax.axis_index("c")
        base = wid * b_per_w

        def body(i, carry):
            off = base + i * chunk
            pltpu.sync_copy(idx_hbm.at[pl.ds(off, chunk)], idx_v)
            pltpu.async_copy(table_hbm.at[idx_v], rows_v, sem).wait()
            pltpu.sync_copy(rows_v, out_hbm.at[pl.ds(off, chunk)])
            return carry

        lax.fori_loop(0, n_chunks, body, 0)

    return gather_kernel(table, idx)


def _gru_body(x_ref, hnei_ref, wz_ref, bz_ref, wr_ref, ur_ref, bu_ref,
              wh_ref, bh_ref, out_ref):
    blk = x_ref.shape[0]
    x = x_ref[...]
    h_nei = hnei_ref[...]
    sum_h = jnp.sum(h_nei, axis=1)
    wz = wz_ref[...]
    z_in = x @ wz[:HIDDEN] + sum_h @ wz[HIDDEN:] + bz_ref[...]
    z = jax.nn.sigmoid(z_in)
    r1 = x @ wr_ref[...]
    r2 = jnp.reshape(jnp.reshape(h_nei, (blk * MAX_NEI, HIDDEN)) @ ur_ref[...],
                     (blk, MAX_NEI, HIDDEN)) + bu_ref[...][None]
    r = jax.nn.sigmoid(r1[:, None, :] + r2)
    sum_gated = jnp.sum(r * h_nei, axis=1)
    wh = wh_ref[...]
    pre_h = jnp.tanh(x @ wh[:HIDDEN] + sum_gated @ wh[HIDDEN:] + bh_ref[...])
    new_h = (1.0 - z) * sum_h + z * pre_h

    @pl.when(pl.program_id(0) == 0)
    def _zero_pad_row():
        row = lax.broadcasted_iota(jnp.int32, (blk, 1), 0)
        out_ref[...] = jnp.where(row == 0, 0.0, new_h)

    @pl.when(pl.program_id(0) != 0)
    def _store():
        out_ref[...] = new_h


def _gru_step(cur_x, h_nei, W_z_w, W_z_b, W_r_w, U_r_w, U_r_b, W_h_w, W_h_b,
              blk):
    e_pad = cur_x.shape[0]
    grid = e_pad // blk

    def full(shape):
        return pl.BlockSpec(shape, lambda i: (0,) * len(shape))

    return pl.pallas_call(
        _gru_body,
        grid=(grid,),
        in_specs=[
            pl.BlockSpec((blk, HIDDEN), lambda i: (i, 0)),
            pl.BlockSpec((blk, MAX_NEI, HIDDEN), lambda i: (i, 0, 0)),
            full((2 * HIDDEN, HIDDEN)),
            full((1, HIDDEN)),
            full((HIDDEN, HIDDEN)),
            full((HIDDEN, HIDDEN)),
            full((1, HIDDEN)),
            full((2 * HIDDEN, HIDDEN)),
            full((1, HIDDEN)),
        ],
        out_specs=pl.BlockSpec((blk, HIDDEN), lambda i: (i, 0)),
        out_shape=jax.ShapeDtypeStruct((e_pad, HIDDEN), jnp.float32),
    )(cur_x, h_nei, W_z_w, W_z_b.reshape(1, HIDDEN), W_r_w, U_r_w,
      U_r_b.reshape(1, HIDDEN), W_h_w, W_h_b.reshape(1, HIDDEN))


def _root_body(x_ref, hnei_ref, w_ref, b_ref, out_ref):
    x = x_ref[...]
    sum_h = jnp.sum(hnei_ref[...], axis=1)
    w = w_ref[...]
    out = x @ w[:HIDDEN] + sum_h @ w[HIDDEN:] + b_ref[...]
    out_ref[...] = jnp.maximum(out, 0.0)


def _root_eval(root_x, root_h_nei, W_w, W_b):
    b = root_x.shape[0]
    return pl.pallas_call(
        _root_body,
        out_shape=jax.ShapeDtypeStruct((b, HIDDEN), jnp.float32),
    )(root_x, root_h_nei, W_w, W_b.reshape(1, HIDDEN))


def kernel(node_wid, fmess, mess_graph, root_idx, root_mess, emb,
           W_z_w, W_z_b, W_r_w, U_r_w, U_r_b, W_h_w, W_h_b, W_w, W_b):
    E = fmess.shape[0]
    BLK = 256
    e_pad = ((E + BLK - 1) // BLK) * BLK

    wid = jnp.take(node_wid, fmess, axis=0).astype(jnp.int32)
    pad_n = e_pad - E
    wid_pad = jnp.concatenate(
        [wid, jnp.arange(pad_n, dtype=jnp.int32) % emb.shape[0]])

    mg_flat = jnp.reshape(mess_graph.astype(jnp.int32), (E * MAX_NEI,))
    mg_pad = jnp.concatenate(
        [mg_flat,
         jnp.arange(pad_n * MAX_NEI, dtype=jnp.int32) % E])

    cur_x = _sc_gather(emb, wid_pad, chunk=224)

    h = jnp.zeros((e_pad, HIDDEN), jnp.float32)
    for _ in range(DEPTH):
        h_nei_flat = _sc_gather(h, mg_pad, chunk=256)
        h_nei = jnp.reshape(h_nei_flat, (e_pad, MAX_NEI, HIDDEN))
        h = _gru_step(cur_x, h_nei, W_z_w, W_z_b, W_r_w, U_r_w, U_r_b,
                      W_h_w, W_h_b, BLK)

    root_wid = jnp.take(node_wid, root_idx, axis=0).astype(jnp.int32)
    root_x = _sc_gather(emb, root_wid, chunk=32)
    rm_flat = jnp.reshape(root_mess.astype(jnp.int32), (-1,))
    root_h_nei_flat = _sc_gather(h, rm_flat, chunk=256)
    root_h_nei = jnp.reshape(root_h_nei_flat,
                             (root_idx.shape[0], MAX_NEI, HIDDEN))
    root_vecs = _root_eval(root_x, root_h_nei, W_w, W_b)

    return (h[:E], root_vecs)


# speedup vs baseline: 2.7577x; 2.7577x over previous
import functools

import jax
import jax.numpy as jnp
from jax import lax
from jax.experimental import pallas as pl
from jax.experimental.pallas import tpu as pltpu
from jax.experimental.pallas import tpu_sc as plsc

HIDDEN = 128
DEPTH = 10
MAX_NEI = 8
NW = 32


@functools.partial(jax.jit, static_argnames=("chunk",))
def _sc_gather(table, idx, *, chunk):
    B = idx.shape[0]
    H = table.shape[1]
    b_per_w = B // NW
    n_chunks = b_per_w // chunk
    mesh = plsc.VectorSubcoreMesh(core_axis_name="c", subcore_axis_name="s")

    @functools.partial(
        pl.kernel,
        out_type=jax.ShapeDtypeStruct((B, H), jnp.float32),
        mesh=mesh,
        scratch_types=[
            pltpu.VMEM((chunk,), jnp.int32),
            pltpu.VMEM((chunk, H), jnp.float32),
            pltpu.SemaphoreType.DMA,
        ],
    )
    def gather_kernel(table_hbm, idx_hbm, out_hbm, idx_v, rows_v, sem):
        wid = lax.axis_index("s") * 2 + lax.axis_index("c")
        base = wid * b_per_w

        def body(i, carry):
            off = base + i * chunk
            pltpu.sync_copy(idx_hbm.at[pl.ds(off, chunk)], idx_v)
            pltpu.async_copy(table_hbm.at[idx_v], rows_v, sem).wait()
            pltpu.sync_copy(rows_v, out_hbm.at[pl.ds(off, chunk)])
            return carry

        lax.fori_loop(0, n_chunks, body, 0)

    return gather_kernel(table, idx)


def _gru_body(x_ref, hnei_ref, wz_ref, bz_ref, wr_ref, ur_ref, bu_ref,
              wh_ref, bh_ref, out_ref):
    blk = x_ref.shape[0]
    x = x_ref[...]
    h_nei = hnei_ref[...]
    sum_h = jnp.sum(h_nei, axis=1)
    wz = wz_ref[...]
    z_in = x @ wz[:HIDDEN] + sum_h @ wz[HIDDEN:] + bz_ref[...]
    z = jax.nn.sigmoid(z_in)
    r1 = x @ wr_ref[...]
    r2 = jnp.reshape(jnp.reshape(h_nei, (blk * MAX_NEI, HIDDEN)) @ ur_ref[...],
                     (blk, MAX_NEI, HIDDEN)) + bu_ref[...][None]
    r = jax.nn.sigmoid(r1[:, None, :] + r2)
    sum_gated = jnp.sum(r * h_nei, axis=1)
    wh = wh_ref[...]
    pre_h = jnp.tanh(x @ wh[:HIDDEN] + sum_gated @ wh[HIDDEN:] + bh_ref[...])
    new_h = (1.0 - z) * sum_h + z * pre_h

    @pl.when(pl.program_id(0) == 0)
    def _zero_pad_row():
        row = lax.broadcasted_iota(jnp.int32, (blk, 1), 0)
        out_ref[...] = jnp.where(row == 0, 0.0, new_h)

    @pl.when(pl.program_id(0) != 0)
    def _store():
        out_ref[...] = new_h


def _gru_step(cur_x, h_nei, W_z_w, W_z_b, W_r_w, U_r_w, U_r_b, W_h_w, W_h_b,
              blk):
    e_pad = cur_x.shape[0]
    grid = e_pad // blk

    def full(shape):
        return pl.BlockSpec(shape, lambda i: (0,) * len(shape))

    return pl.pallas_call(
        _gru_body,
        grid=(grid,),
        in_specs=[
            pl.BlockSpec((blk, HIDDEN), lambda i: (i, 0)),
            pl.BlockSpec((blk, MAX_NEI, HIDDEN), lambda i: (i, 0, 0)),
            full((2 * HIDDEN, HIDDEN)),
            full((1, HIDDEN)),
            full((HIDDEN, HIDDEN)),
            full((HIDDEN, HIDDEN)),
            full((1, HIDDEN)),
            full((2 * HIDDEN, HIDDEN)),
            full((1, HIDDEN)),
        ],
        out_specs=pl.BlockSpec((blk, HIDDEN), lambda i: (i, 0)),
        out_shape=jax.ShapeDtypeStruct((e_pad, HIDDEN), jnp.float32),
    )(cur_x, h_nei, W_z_w, W_z_b.reshape(1, HIDDEN), W_r_w, U_r_w,
      U_r_b.reshape(1, HIDDEN), W_h_w, W_h_b.reshape(1, HIDDEN))


def _root_body(x_ref, hnei_ref, w_ref, b_ref, out_ref):
    x = x_ref[...]
    sum_h = jnp.sum(hnei_ref[...], axis=1)
    w = w_ref[...]
    out = x @ w[:HIDDEN] + sum_h @ w[HIDDEN:] + b_ref[...]
    out_ref[...] = jnp.maximum(out, 0.0)


def _root_eval(root_x, root_h_nei, W_w, W_b):
    b = root_x.shape[0]
    return pl.pallas_call(
        _root_body,
        out_shape=jax.ShapeDtypeStruct((b, HIDDEN), jnp.float32),
    )(root_x, root_h_nei, W_w, W_b.reshape(1, HIDDEN))


def kernel(node_wid, fmess, mess_graph, root_idx, root_mess, emb,
           W_z_w, W_z_b, W_r_w, U_r_w, U_r_b, W_h_w, W_h_b, W_w, W_b):
    E = fmess.shape[0]
    BLK = 256
    e_pad = ((E + BLK - 1) // BLK) * BLK

    wid = jnp.take(node_wid, fmess, axis=0).astype(jnp.int32)
    pad_n = e_pad - E
    wid_pad = jnp.concatenate(
        [wid, jnp.arange(pad_n, dtype=jnp.int32) % emb.shape[0]])

    mg_flat = jnp.reshape(mess_graph.astype(jnp.int32), (E * MAX_NEI,))
    mg_pad = jnp.concatenate(
        [mg_flat,
         jnp.arange(pad_n * MAX_NEI, dtype=jnp.int32) % E])

    cur_x = _sc_gather(emb, wid_pad, chunk=112)

    h = jnp.zeros((e_pad, HIDDEN), jnp.float32)
    for _ in range(DEPTH):
        h_nei_flat = _sc_gather(h, mg_pad, chunk=128)
        h_nei = jnp.reshape(h_nei_flat, (e_pad, MAX_NEI, HIDDEN))
        h = _gru_step(cur_x, h_nei, W_z_w, W_z_b, W_r_w, U_r_w, U_r_b,
                      W_h_w, W_h_b, BLK)

    root_wid = jnp.take(node_wid, root_idx, axis=0).astype(jnp.int32)
    root_x = _sc_gather(emb, root_wid, chunk=32)
    rm_flat = jnp.reshape(root_mess.astype(jnp.int32), (-1,))
    root_h_nei_flat = _sc_gather(h, rm_flat, chunk=128)
    root_h_nei = jnp.reshape(root_h_nei_flat,
                             (root_idx.shape[0], MAX_NEI, HIDDEN))
    root_vecs = _root_eval(root_x, root_h_nei, W_w, W_b)

    return (h[:E], root_vecs)


# speedup vs baseline: 3.6822x; 1.3353x over previous
import functools

import jax
import jax.numpy as jnp
from jax import lax
from jax.experimental import pallas as pl
from jax.experimental.pallas import tpu as pltpu
from jax.experimental.pallas import tpu_sc as plsc

HIDDEN = 128
DEPTH = 10
MAX_NEI = 8
NW = 32
NBUF = 4


@functools.partial(jax.jit, static_argnames=("chunk",))
def _sc_gather(table, idx, *, chunk):
    B = idx.shape[0]
    H = table.shape[1]
    b_per_w = B // NW
    n_chunks = b_per_w // chunk
    n_outer = n_chunks // NBUF
    mesh = plsc.VectorSubcoreMesh(core_axis_name="c", subcore_axis_name="s")

    @functools.partial(
        pl.kernel,
        out_type=jax.ShapeDtypeStruct((B, H), jnp.float32),
        mesh=mesh,
        scratch_types=(
            [pltpu.VMEM((b_per_w,), jnp.int32)]
            + [pltpu.VMEM((chunk, H), jnp.float32) for _ in range(NBUF)]
            + [pltpu.SemaphoreType.DMA for _ in range(2 * NBUF)]
        ),
    )
    def gather_kernel(table_hbm, idx_hbm, out_hbm, idx_all, *bufs):
        rows = bufs[:NBUF]
        gsem = bufs[NBUF:2 * NBUF]
        wsem = bufs[2 * NBUF:]
        wid = lax.axis_index("s") * 2 + lax.axis_index("c")
        base = wid * b_per_w
        pltpu.sync_copy(idx_hbm.at[pl.ds(base, b_per_w)], idx_all)

        def body(j, carry):
            offs = [(j * NBUF + s) * chunk for s in range(NBUF)]
            for s in range(NBUF):
                @pl.when(j > 0)
                def _wait_wb(s=s):
                    pltpu.make_async_copy(
                        rows[s], out_hbm.at[pl.ds(base, chunk)], wsem[s]
                    ).wait()
                pltpu.async_copy(
                    table_hbm.at[idx_all.at[pl.ds(offs[s], chunk)]],
                    rows[s], gsem[s])
            for s in range(NBUF):
                pltpu.make_async_copy(
                    table_hbm.at[idx_all.at[pl.ds(offs[s], chunk)]],
                    rows[s], gsem[s]).wait()
                pltpu.async_copy(rows[s],
                                 out_hbm.at[pl.ds(base + offs[s], chunk)],
                                 wsem[s])
            return carry

        lax.fori_loop(0, n_outer, body, 0)
        for s in range(NBUF):
            pltpu.make_async_copy(
                rows[s], out_hbm.at[pl.ds(base, chunk)], wsem[s]).wait()

    return gather_kernel(table, idx)


def _pre_body(x_ref, wzt_ref, bz_ref, wr_ref, bu_ref, wht_ref, bh_ref,
              xz_ref, xr_ref, xh_ref, h1_ref):
    x = x_ref[...]
    xz = x @ wzt_ref[...] + bz_ref[...]
    xr = x @ wr_ref[...] + bu_ref[...]
    xh = x @ wht_ref[...] + bh_ref[...]
    xz_ref[...] = xz
    xr_ref[...] = xr
    xh_ref[...] = xh
    h1 = jax.nn.sigmoid(xz) * jnp.tanh(xh)

    @pl.when(pl.program_id(0) == 0)
    def _zero_pad_row():
        row = lax.broadcasted_iota(jnp.int32, (x_ref.shape[0], 1), 0)
        h1_ref[...] = jnp.where(row == 0, 0.0, h1)

    @pl.when(pl.program_id(0) != 0)
    def _store():
        h1_ref[...] = h1


def _precompute(cur_x, W_z_w, W_z_b, W_r_w, U_r_b, W_h_w, W_h_b, blk):
    e_pad = cur_x.shape[0]
    grid = e_pad // blk

    def full(shape):
        return pl.BlockSpec(shape, lambda i: (0,) * len(shape))

    row_spec = pl.BlockSpec((blk, HIDDEN), lambda i: (i, 0))
    out = jax.ShapeDtypeStruct((e_pad, HIDDEN), jnp.float32)
    return pl.pallas_call(
        _pre_body,
        grid=(grid,),
        in_specs=[row_spec] + [full((HIDDEN, HIDDEN)), full((1, HIDDEN))] * 3,
        out_specs=[row_spec] * 4,
        out_shape=[out] * 4,
    )(cur_x, W_z_w[:HIDDEN], W_z_b.reshape(1, HIDDEN), W_r_w,
      U_r_b.reshape(1, HIDDEN), W_h_w[:HIDDEN], W_h_b.reshape(1, HIDDEN))


def _gru_body(xz_ref, xr_ref, xh_ref, hnei_ref, wzb_ref, ur_ref, whb_ref,
              out_ref):
    blk = xz_ref.shape[0]
    h_nei = hnei_ref[...]
    sum_h = jnp.sum(h_nei, axis=1)
    z = jax.nn.sigmoid(xz_ref[...] + sum_h @ wzb_ref[...])
    r2 = jnp.reshape(jnp.reshape(h_nei, (blk * MAX_NEI, HIDDEN)) @ ur_ref[...],
                     (blk, MAX_NEI, HIDDEN))
    r = jax.nn.sigmoid(xr_ref[...][:, None, :] + r2)
    sum_gated = jnp.sum(r * h_nei, axis=1)
    pre_h = jnp.tanh(xh_ref[...] + sum_gated @ whb_ref[...])
    new_h = (1.0 - z) * sum_h + z * pre_h

    @pl.when(pl.program_id(0) == 0)
    def _zero_pad_row():
        row = lax.broadcasted_iota(jnp.int32, (blk, 1), 0)
        out_ref[...] = jnp.where(row == 0, 0.0, new_h)

    @pl.when(pl.program_id(0) != 0)
    def _store():
        out_ref[...] = new_h


def _gru_step(xz, xr, xh, h_nei, wz_bot, U_r_w, wh_bot, blk):
    e_pad = xz.shape[0]
    grid = e_pad // blk

    def full(shape):
        return pl.BlockSpec(shape, lambda i: (0,) * len(shape))

    row_spec = pl.BlockSpec((blk, HIDDEN), lambda i: (i, 0))
    return pl.pallas_call(
        _gru_body,
        grid=(grid,),
        in_specs=[
            row_spec, row_spec, row_spec,
            pl.BlockSpec((blk, MAX_NEI, HIDDEN), lambda i: (i, 0, 0)),
            full((HIDDEN, HIDDEN)),
            full((HIDDEN, HIDDEN)),
            full((HIDDEN, HIDDEN)),
        ],
        out_specs=row_spec,
        out_shape=jax.ShapeDtypeStruct((e_pad, HIDDEN), jnp.float32),
    )(xz, xr, xh, h_nei, wz_bot, U_r_w, wh_bot)


def _root_body(x_ref, hnei_ref, w_ref, b_ref, out_ref):
    x = x_ref[...]
    sum_h = jnp.sum(hnei_ref[...], axis=1)
    w = w_ref[...]
    out = x @ w[:HIDDEN] + sum_h @ w[HIDDEN:] + b_ref[...]
    out_ref[...] = jnp.maximum(out, 0.0)


def _root_eval(root_x, root_h_nei, W_w, W_b):
    b = root_x.shape[0]
    return pl.pallas_call(
        _root_body,
        out_shape=jax.ShapeDtypeStruct((b, HIDDEN), jnp.float32),
    )(root_x, root_h_nei, W_w, W_b.reshape(1, HIDDEN))


def kernel(node_wid, fmess, mess_graph, root_idx, root_mess, emb,
           W_z_w, W_z_b, W_r_w, U_r_w, U_r_b, W_h_w, W_h_b, W_w, W_b):
    E = fmess.shape[0]
    BLK = 256
    e_pad = ((E + BLK - 1) // BLK) * BLK

    wid = jnp.take(node_wid, fmess, axis=0).astype(jnp.int32)
    pad_n = e_pad - E
    wid_pad = jnp.concatenate(
        [wid, jnp.arange(pad_n, dtype=jnp.int32) % emb.shape[0]])

    mg_flat = jnp.reshape(mess_graph.astype(jnp.int32), (E * MAX_NEI,))
    mg_pad = jnp.concatenate(
        [mg_flat, jnp.arange(pad_n * MAX_NEI, dtype=jnp.int32) % E])

    cur_x = _sc_gather(emb, wid_pad, chunk=112)
    xz, xr, xh, h = _precompute(cur_x, W_z_w, W_z_b, W_r_w, U_r_b,
                                W_h_w, W_h_b, BLK)
    wz_bot = W_z_w[HIDDEN:]
    wh_bot = W_h_w[HIDDEN:]

    for _ in range(DEPTH - 1):
        h_nei_flat = _sc_gather(h, mg_pad, chunk=128)
        h_nei = jnp.reshape(h_nei_flat, (e_pad, MAX_NEI, HIDDEN))
        h = _gru_step(xz, xr, xh, h_nei, wz_bot, U_r_w, wh_bot, BLK)

    root_wid = jnp.take(node_wid, root_idx, axis=0).astype(jnp.int32)
    root_x = _sc_gather(emb, root_wid, chunk=8)
    rm_flat = jnp.reshape(root_mess.astype(jnp.int32), (-1,))
    root_h_nei_flat = _sc_gather(h, rm_flat, chunk=64)
    root_h_nei = jnp.reshape(root_h_nei_flat,
                             (root_idx.shape[0], MAX_NEI, HIDDEN))
    root_vecs = _root_eval(root_x, root_h_nei, W_w, W_b)

    return (h[:E], root_vecs)


# speedup vs baseline: 3.9675x; 1.0775x over previous
import functools

import jax
import jax.numpy as jnp
from jax import lax
from jax.experimental import pallas as pl
from jax.experimental.pallas import tpu as pltpu
from jax.experimental.pallas import tpu_sc as plsc

HIDDEN = 128
DEPTH = 10
MAX_NEI = 8
NW = 32
NBUF = 4


@functools.partial(jax.jit, static_argnames=("chunk",))
def _sc_gather(table, idx, *, chunk):
    B = idx.shape[0]
    H = table.shape[1]
    dtype = table.dtype
    b_per_w = B // NW
    n_chunks = b_per_w // chunk
    n_outer = n_chunks // NBUF
    assert chunk <= 128 and chunk % 8 == 0, chunk
    assert B == NW * n_outer * NBUF * chunk, (B, chunk)
    assert n_chunks % 8 == 0, (B, chunk)
    mesh = plsc.VectorSubcoreMesh(core_axis_name="c", subcore_axis_name="s")

    @functools.partial(
        pl.kernel,
        out_type=jax.ShapeDtypeStruct((B, H), dtype),
        mesh=mesh,
        scratch_types=(
            [pltpu.VMEM((n_chunks, chunk), jnp.int32)]
            + [pltpu.VMEM((chunk, H), dtype) for _ in range(NBUF)]
            + [pltpu.SemaphoreType.DMA for _ in range(2 * NBUF)]
        ),
    )
    def gather_kernel(table_hbm, idx_hbm, out_hbm, idx_2d, *bufs):
        rows = bufs[:NBUF]
        gsem = bufs[NBUF:2 * NBUF]
        wsem = bufs[2 * NBUF:]
        wid = lax.axis_index("s") * 2 + lax.axis_index("c")
        base = wid * b_per_w
        pltpu.sync_copy(idx_hbm.at[pl.ds(wid * n_chunks, n_chunks)], idx_2d)

        def body(j, carry):
            for s in range(NBUF):
                @pl.when(j > 0)
                def _wait_wb(s=s):
                    pltpu.make_async_copy(
                        rows[s], out_hbm.at[pl.ds(base, chunk)], wsem[s]
                    ).wait()
                pltpu.async_copy(
                    table_hbm.at[idx_2d.at[j * NBUF + s]],
                    rows[s], gsem[s])
            for s in range(NBUF):
                pltpu.make_async_copy(
                    table_hbm.at[idx_2d.at[j * NBUF + s]],
                    rows[s], gsem[s]).wait()
                pltpu.async_copy(
                    rows[s],
                    out_hbm.at[pl.ds(base + (j * NBUF + s) * chunk, chunk)],
                    wsem[s])
            return carry

        lax.fori_loop(0, n_outer, body, 0)
        for s in range(NBUF):
            pltpu.make_async_copy(
                rows[s], out_hbm.at[pl.ds(base, chunk)], wsem[s]).wait()

    return gather_kernel(table, idx.reshape(-1, chunk))


def _pre_body(x_ref, wzt_ref, bz_ref, wr_ref, bu_ref, wht_ref, bh_ref,
              xz_ref, xr_ref, xh_ref, h1_ref):
    x = x_ref[...]
    xz = x @ wzt_ref[...] + bz_ref[...]
    xr = x @ wr_ref[...] + bu_ref[...]
    xh = x @ wht_ref[...] + bh_ref[...]
    xz_ref[...] = xz
    xr_ref[...] = xr
    xh_ref[...] = xh
    h1 = jax.nn.sigmoid(xz) * jnp.tanh(xh)

    @pl.when(pl.program_id(0) == 0)
    def _zero_pad_row():
        row = lax.broadcasted_iota(jnp.int32, (x_ref.shape[0], 1), 0)
        h1_ref[...] = jnp.where(row == 0, 0.0, h1)

    @pl.when(pl.program_id(0) != 0)
    def _store():
        h1_ref[...] = h1


def _precompute(cur_x, W_z_w, W_z_b, W_r_w, U_r_b, W_h_w, W_h_b, blk):
    e_pad = cur_x.shape[0]
    grid = e_pad // blk

    def full(shape):
        return pl.BlockSpec(shape, lambda i: (0,) * len(shape))

    row_spec = pl.BlockSpec((blk, HIDDEN), lambda i: (i, 0))
    out = jax.ShapeDtypeStruct((e_pad, HIDDEN), jnp.float32)
    return pl.pallas_call(
        _pre_body,
        grid=(grid,),
        in_specs=[row_spec] + [full((HIDDEN, HIDDEN)), full((1, HIDDEN))] * 3,
        out_specs=[row_spec] * 4,
        out_shape=[out] * 4,
    )(cur_x, W_z_w[:HIDDEN], W_z_b.reshape(1, HIDDEN), W_r_w,
      U_r_b.reshape(1, HIDDEN), W_h_w[:HIDDEN], W_h_b.reshape(1, HIDDEN))


def _gru_body(xz_ref, xr_ref, xh_ref, hnei_ref, wzb_ref, ur_ref, whb_ref,
              out_ref, *, zero_row):
    blk = xz_ref.shape[0]
    h_nei = hnei_ref[...]
    sum_h = jnp.sum(h_nei, axis=1)
    z = jax.nn.sigmoid(xz_ref[...] + sum_h @ wzb_ref[...])
    r2 = jnp.reshape(jnp.reshape(h_nei, (blk * MAX_NEI, HIDDEN)) @ ur_ref[...],
                     (blk, MAX_NEI, HIDDEN))
    r = jax.nn.sigmoid(xr_ref[...][:, None, :] + r2)
    sum_gated = jnp.sum(r * h_nei, axis=1)
    pre_h = jnp.tanh(xh_ref[...] + sum_gated @ whb_ref[...])
    new_h = (1.0 - z) * sum_h + z * pre_h

    if zero_row:
        @pl.when(pl.program_id(0) == 0)
        def _zero_pad_row():
            row = lax.broadcasted_iota(jnp.int32, (blk, 1), 0)
            out_ref[...] = jnp.where(row == 0, 0.0, new_h)

        @pl.when(pl.program_id(0) != 0)
        def _store():
            out_ref[...] = new_h
    else:
        out_ref[...] = new_h


def _gru_step(xz, xr, xh, h_nei, wz_bot, ur_w, wh_bot, blk, zero_row):
    e_pad = xz.shape[0]
    blk = min(blk, e_pad)
    grid = e_pad // blk

    def full(shape):
        return pl.BlockSpec(shape, lambda i: (0,) * len(shape))

    row_spec = pl.BlockSpec((blk, HIDDEN), lambda i: (i, 0))
    return pl.pallas_call(
        functools.partial(_gru_body, zero_row=zero_row),
        grid=(grid,),
        in_specs=[
            row_spec, row_spec, row_spec,
            pl.BlockSpec((blk, MAX_NEI, HIDDEN), lambda i: (i, 0, 0)),
            full((HIDDEN, HIDDEN)),
            full((HIDDEN, HIDDEN)),
            full((HIDDEN, HIDDEN)),
        ],
        out_specs=row_spec,
        out_shape=jax.ShapeDtypeStruct((e_pad, HIDDEN), jnp.float32),
    )(xz, xr, xh, h_nei, wz_bot, ur_w, wh_bot)


def _root_body(x_ref, hnei_ref, w_ref, b_ref, out_ref):
    x = x_ref[...]
    sum_h = jnp.sum(hnei_ref[...], axis=1)
    w = w_ref[...]
    out = x @ w[:HIDDEN] + sum_h @ w[HIDDEN:] + b_ref[...]
    out_ref[...] = jnp.maximum(out, 0.0)


def _root_eval(root_x, root_h_nei, W_w, W_b):
    b = root_x.shape[0]
    return pl.pallas_call(
        _root_body,
        out_shape=jax.ShapeDtypeStruct((b, HIDDEN), jnp.float32),
    )(root_x, root_h_nei, W_w, W_b.reshape(1, HIDDEN))


def kernel(node_wid, fmess, mess_graph, root_idx, root_mess, emb,
           W_z_w, W_z_b, W_r_w, U_r_w, U_r_b, W_h_w, W_h_b, W_w, W_b):
    E = fmess.shape[0]
    BLK = 256
    e_pad = ((E + 14335) // 14336) * 14336

    pad_n = e_pad - E
    N = node_wid.shape[0]
    fm_pad = jnp.concatenate(
        [fmess.astype(jnp.int32), jnp.arange(pad_n, dtype=jnp.int32) % N])

    mg_flat = jnp.reshape(mess_graph.astype(jnp.int32), (E * MAX_NEI,))
    mg_pad = jnp.concatenate(
        [mg_flat, jnp.arange(pad_n * MAX_NEI, dtype=jnp.int32) % E])

    n_pad = ((N + 2047) // 2048) * 2048
    nw_pad = jnp.concatenate(
        [node_wid.astype(jnp.int32),
         jnp.arange(n_pad - N, dtype=jnp.int32) % emb.shape[0]])
    node_x = _sc_gather(emb, nw_pad, chunk=40)
    cur_x = _sc_gather(node_x, fm_pad, chunk=56)
    xz, xr, xh, h = _precompute(cur_x, W_z_w, W_z_b, W_r_w, U_r_b,
                                W_h_w, W_h_b, BLK)
    wz_bot = W_z_w[HIDDEN:]
    wh_bot = W_h_w[HIDDEN:]

    EH = e_pad // 2
    mgA, mgB = mg_pad[:EH * MAX_NEI], mg_pad[EH * MAX_NEI:]
    xzA, xzB = xz[:EH], xz[EH:]
    xrA, xrB = xr[:EH], xr[EH:]
    xhA, xhB = xh[:EH], xh[EH:]

    for _ in range(DEPTH - 1):
        gA = _sc_gather(h, mgA, chunk=112)
        gB = _sc_gather(h, mgB, chunk=112)
        hA = _gru_step(xzA, xrA, xhA,
                       jnp.reshape(gA, (EH, MAX_NEI, HIDDEN)),
                       wz_bot, U_r_w, wh_bot, BLK, True)
        hB = _gru_step(xzB, xrB, xhB,
                       jnp.reshape(gB, (EH, MAX_NEI, HIDDEN)),
                       wz_bot, U_r_w, wh_bot, BLK, False)
        h = jnp.concatenate([hA, hB])

    B_r = root_idx.shape[0]
    ri_pad = jnp.concatenate(
        [root_idx.astype(jnp.int32),
         jnp.arange(B_r, dtype=jnp.int32) % N])
    root_x = _sc_gather(node_x, ri_pad, chunk=8)[:B_r]
    rm_flat = jnp.reshape(root_mess.astype(jnp.int32), (-1,))
    root_h_nei_flat = _sc_gather(h, rm_flat, chunk=32)
    root_h_nei = jnp.reshape(root_h_nei_flat,
                             (root_idx.shape[0], MAX_NEI, HIDDEN))
    root_vecs = _root_eval(root_x, root_h_nei, W_w, W_b)

    return (h[:E], root_vecs)


# speedup vs baseline: 4.1605x; 1.0487x over previous
import functools

import jax
import jax.numpy as jnp
from jax import lax
from jax.experimental import pallas as pl
from jax.experimental.pallas import tpu as pltpu
from jax.experimental.pallas import tpu_sc as plsc

HIDDEN = 128
DEPTH = 10
MAX_NEI = 8
NW = 32
NBUF = 4


@functools.partial(jax.jit, static_argnames=("chunk",))
def _sc_gather(table, idx, *, chunk):
    B = idx.shape[0]
    H = table.shape[1]
    dtype = table.dtype
    b_per_w = B // NW
    n_chunks = b_per_w // chunk
    n_outer = n_chunks // NBUF
    assert chunk <= 128 and chunk % 8 == 0, chunk
    assert B == NW * n_outer * NBUF * chunk, (B, chunk)
    mesh = plsc.VectorSubcoreMesh(core_axis_name="c", subcore_axis_name="s")

    @functools.partial(
        pl.kernel,
        out_type=jax.ShapeDtypeStruct((B, H), dtype),
        mesh=mesh,
        scratch_types=(
            [pltpu.VMEM((b_per_w,), jnp.int32)]
            + [pltpu.VMEM((chunk, H), dtype) for _ in range(NBUF)]
            + [pltpu.SemaphoreType.DMA for _ in range(2 * NBUF)]
        ),
    )
    def gather_kernel(table_hbm, idx_hbm, out_hbm, idx_all, *bufs):
        rows = bufs[:NBUF]
        gsem = bufs[NBUF:2 * NBUF]
        wsem = bufs[2 * NBUF:]
        wid = lax.axis_index("s") * 2 + lax.axis_index("c")
        base = wid * b_per_w
        pltpu.sync_copy(idx_hbm.at[pl.ds(base, b_per_w)], idx_all)

        def body(j, carry):
            for s in range(NBUF):
                @pl.when(j > 0)
                def _wait_wb(s=s):
                    pltpu.make_async_copy(
                        rows[s], out_hbm.at[pl.ds(base, chunk)], wsem[s]
                    ).wait()
                pltpu.async_copy(
                    table_hbm.at[idx_all.at[pl.ds((j * NBUF + s) * chunk,
                                                  chunk)]],
                    rows[s], gsem[s])
            for s in range(NBUF):
                pltpu.make_async_copy(
                    table_hbm.at[idx_all.at[pl.ds((j * NBUF + s) * chunk,
                                                  chunk)]],
                    rows[s], gsem[s]).wait()
                pltpu.async_copy(
                    rows[s],
                    out_hbm.at[pl.ds(base + (j * NBUF + s) * chunk, chunk)],
                    wsem[s])
            return carry

        lax.fori_loop(0, n_outer, body, 0)
        for s in range(NBUF):
            pltpu.make_async_copy(
                rows[s], out_hbm.at[pl.ds(base, chunk)], wsem[s]).wait()

    return gather_kernel(table, idx)


def _pre_body(x_ref, wzt_ref, bz_ref, wr_ref, bu_ref, wht_ref, bh_ref,
              xz_ref, xr_ref, xh_ref, h1_ref):
    x = x_ref[...]
    xz = x @ wzt_ref[...] + bz_ref[...]
    xr = x @ wr_ref[...] + bu_ref[...]
    xh = x @ wht_ref[...] + bh_ref[...]
    xz_ref[...] = xz
    xr_ref[...] = xr
    xh_ref[...] = xh
    h1 = jax.nn.sigmoid(xz) * jnp.tanh(xh)

    @pl.when(pl.program_id(0) == 0)
    def _zero_pad_row():
        row = lax.broadcasted_iota(jnp.int32, (x_ref.shape[0], 1), 0)
        h1_ref[...] = jnp.where(row == 0, 0.0, h1)

    @pl.when(pl.program_id(0) != 0)
    def _store():
        h1_ref[...] = h1


def _precompute(cur_x, W_z_w, W_z_b, W_r_w, U_r_b, W_h_w, W_h_b, blk):
    e_pad = cur_x.shape[0]
    grid = e_pad // blk

    def full(shape):
        return pl.BlockSpec(shape, lambda i: (0,) * len(shape))

    row_spec = pl.BlockSpec((blk, HIDDEN), lambda i: (i, 0))
    out = jax.ShapeDtypeStruct((e_pad, HIDDEN), jnp.float32)
    return pl.pallas_call(
        _pre_body,
        grid=(grid,),
        in_specs=[row_spec] + [full((HIDDEN, HIDDEN)), full((1, HIDDEN))] * 3,
        out_specs=[row_spec] * 4,
        out_shape=[out] * 4,
    )(cur_x, W_z_w[:HIDDEN], W_z_b.reshape(1, HIDDEN), W_r_w,
      U_r_b.reshape(1, HIDDEN), W_h_w[:HIDDEN], W_h_b.reshape(1, HIDDEN))


def _gru_body(xz_ref, xr_ref, xh_ref, hnei_ref, wzb_ref, ur_ref, whb_ref,
              *tail, zero_row):
    out_ref = tail[-1]
    blk = xz_ref.shape[0]
    h_nei = hnei_ref[...]
    sum_h = jnp.sum(h_nei, axis=1)
    z = jax.nn.sigmoid(xz_ref[...] + sum_h @ wzb_ref[...])
    r2 = jnp.reshape(jnp.reshape(h_nei, (blk * MAX_NEI, HIDDEN)) @ ur_ref[...],
                     (blk, MAX_NEI, HIDDEN))
    r = jax.nn.sigmoid(xr_ref[...][:, None, :] + r2)
    sum_gated = jnp.sum(r * h_nei, axis=1)
    pre_h = jnp.tanh(xh_ref[...] + sum_gated @ whb_ref[...])
    new_h = (1.0 - z) * sum_h + z * pre_h

    if zero_row:
        @pl.when(pl.program_id(0) == 0)
        def _zero_pad_row():
            row = lax.broadcasted_iota(jnp.int32, (blk, 1), 0)
            out_ref[...] = jnp.where(row == 0, 0.0, new_h)

        @pl.when(pl.program_id(0) != 0)
        def _store():
            out_ref[...] = new_h
    else:
        out_ref[...] = new_h


def _gru_step(xz, xr, xh, h_nei, wz_bot, ur_w, wh_bot, blk, zero_row,
              e_full, off_blocks=0, out_buf=None):
    n_e = xz.shape[0]
    blk = min(blk, n_e)
    grid = n_e // blk

    def full(shape):
        return pl.BlockSpec(shape, lambda i: (0,) * len(shape))

    row_spec = pl.BlockSpec((blk, HIDDEN), lambda i: (i, 0))
    in_specs = [
        row_spec, row_spec, row_spec,
        pl.BlockSpec((blk, MAX_NEI, HIDDEN), lambda i: (i, 0, 0)),
        full((HIDDEN, HIDDEN)),
        full((HIDDEN, HIDDEN)),
        full((HIDDEN, HIDDEN)),
    ]
    args = [xz, xr, xh, h_nei, wz_bot, ur_w, wh_bot]
    aliases = {}
    if out_buf is not None:
        in_specs.append(pl.BlockSpec(memory_space=pl.ANY))
        args.append(out_buf)
        aliases = {7: 0}
    return pl.pallas_call(
        functools.partial(_gru_body, zero_row=zero_row),
        grid=(grid,),
        in_specs=in_specs,
        out_specs=pl.BlockSpec((blk, HIDDEN), lambda i: (i + off_blocks, 0)),
        out_shape=jax.ShapeDtypeStruct((e_full, HIDDEN), jnp.float32),
        input_output_aliases=aliases,
    )(*args)


def _root_body(x_ref, hnei_ref, w_ref, b_ref, out_ref):
    x = x_ref[...]
    sum_h = jnp.sum(hnei_ref[...], axis=1)
    w = w_ref[...]
    out = x @ w[:HIDDEN] + sum_h @ w[HIDDEN:] + b_ref[...]
    out_ref[...] = jnp.maximum(out, 0.0)


def _root_eval(root_x, root_h_nei, W_w, W_b):
    b = root_x.shape[0]
    return pl.pallas_call(
        _root_body,
        out_shape=jax.ShapeDtypeStruct((b, HIDDEN), jnp.float32),
    )(root_x, root_h_nei, W_w, W_b.reshape(1, HIDDEN))


def kernel(node_wid, fmess, mess_graph, root_idx, root_mess, emb,
           W_z_w, W_z_b, W_r_w, U_r_w, U_r_b, W_h_w, W_h_b, W_w, W_b):
    E = fmess.shape[0]
    BLK = 256
    e_pad = ((E + 14335) // 14336) * 14336

    pad_n = e_pad - E
    N = node_wid.shape[0]
    fm_pad = jnp.concatenate(
        [fmess.astype(jnp.int32), jnp.arange(pad_n, dtype=jnp.int32) % N])

    mg_flat = jnp.reshape(mess_graph.astype(jnp.int32), (E * MAX_NEI,))
    mg_pad = jnp.concatenate(
        [mg_flat, jnp.arange(pad_n * MAX_NEI, dtype=jnp.int32) % E])

    n_pad = ((N + 2047) // 2048) * 2048
    nw_pad = jnp.concatenate(
        [node_wid.astype(jnp.int32),
         jnp.arange(n_pad - N, dtype=jnp.int32) % emb.shape[0]])
    node_x = _sc_gather(emb, nw_pad, chunk=80)
    cur_x = _sc_gather(node_x, fm_pad, chunk=112)
    xz, xr, xh, h = _precompute(cur_x, W_z_w, W_z_b, W_r_w, U_r_b,
                                W_h_w, W_h_b, BLK)
    wz_bot = W_z_w[HIDDEN:]
    wh_bot = W_h_w[HIDDEN:]

    EA = e_pad // 4
    mgA, mgB = mg_pad[:EA * MAX_NEI], mg_pad[EA * MAX_NEI:]
    xzA, xzB = xz[:EA], xz[EA:]
    xrA, xrB = xr[:EA], xr[EA:]
    xhA, xhB = xh[:EA], xh[EA:]
    EB = e_pad - EA

    for _ in range(DEPTH - 1):
        gA = _sc_gather(h, mgA, chunk=112)
        gB = _sc_gather(h, mgB, chunk=112)
        hA = _gru_step(xzA, xrA, xhA,
                       jnp.reshape(gA, (EA, MAX_NEI, HIDDEN)),
                       wz_bot, U_r_w, wh_bot, BLK, True, e_pad)
        h = _gru_step(xzB, xrB, xhB,
                      jnp.reshape(gB, (EB, MAX_NEI, HIDDEN)),
                      wz_bot, U_r_w, wh_bot, BLK, False, e_pad,
                      off_blocks=EA // min(BLK, EB), out_buf=hA)

    B_r = root_idx.shape[0]
    ri_pad = jnp.concatenate(
        [root_idx.astype(jnp.int32),
         jnp.arange(B_r, dtype=jnp.int32) % N])
    root_x = _sc_gather(node_x, ri_pad, chunk=8)[:B_r]
    rm_flat = jnp.reshape(root_mess.astype(jnp.int32), (-1,))
    root_h_nei_flat = _sc_gather(h, rm_flat, chunk=32)
    root_h_nei = jnp.reshape(root_h_nei_flat,
                             (root_idx.shape[0], MAX_NEI, HIDDEN))
    root_vecs = _root_eval(root_x, root_h_nei, W_w, W_b)

    return (h[:E], root_vecs)


# speedup vs baseline: 4.5463x; 1.0927x over previous
import functools

import jax
import jax.numpy as jnp
from jax import lax
from jax.experimental import pallas as pl
from jax.experimental.pallas import tpu as pltpu
from jax.experimental.pallas import tpu_sc as plsc

HIDDEN = 128
DEPTH = 10
MAX_NEI = 8
NW = 32
NBUF = 4


@functools.partial(jax.jit, static_argnames=("chunk",))
def _sc_gather(table, idx, *, chunk):
    B = idx.shape[0]
    H = table.shape[1]
    dtype = table.dtype
    b_per_w = B // NW
    n_chunks = b_per_w // chunk
    n_outer = n_chunks // NBUF
    assert chunk <= 128 and chunk % 8 == 0, chunk
    assert B == NW * n_outer * NBUF * chunk, (B, chunk)
    mesh = plsc.VectorSubcoreMesh(core_axis_name="c", subcore_axis_name="s")

    @functools.partial(
        pl.kernel,
        out_type=jax.ShapeDtypeStruct((B, H), dtype),
        mesh=mesh,
        scratch_types=(
            [pltpu.VMEM((b_per_w,), jnp.int32)]
            + [pltpu.VMEM((chunk, H), dtype) for _ in range(NBUF)]
            + [pltpu.SemaphoreType.DMA for _ in range(2 * NBUF)]
        ),
    )
    def gather_kernel(table_hbm, idx_hbm, out_hbm, idx_all, *bufs):
        rows = bufs[:NBUF]
        gsem = bufs[NBUF:2 * NBUF]
        wsem = bufs[2 * NBUF:]
        wid = lax.axis_index("s") * 2 + lax.axis_index("c")
        base = wid * b_per_w
        pltpu.sync_copy(idx_hbm.at[pl.ds(base, b_per_w)], idx_all)

        def body(j, carry):
            for s in range(NBUF):
                @pl.when(j > 0)
                def _wait_wb(s=s):
                    pltpu.make_async_copy(
                        rows[s], out_hbm.at[pl.ds(base, chunk)], wsem[s]
                    ).wait()
                pltpu.async_copy(
                    table_hbm.at[idx_all.at[pl.ds((j * NBUF + s) * chunk,
                                                  chunk)]],
                    rows[s], gsem[s])
            for s in range(NBUF):
                pltpu.make_async_copy(
                    table_hbm.at[idx_all.at[pl.ds((j * NBUF + s) * chunk,
                                                  chunk)]],
                    rows[s], gsem[s]).wait()
                pltpu.async_copy(
                    rows[s],
                    out_hbm.at[pl.ds(base + (j * NBUF + s) * chunk, chunk)],
                    wsem[s])
            return carry

        lax.fori_loop(0, n_outer, body, 0)
        for s in range(NBUF):
            pltpu.make_async_copy(
                rows[s], out_hbm.at[pl.ds(base, chunk)], wsem[s]).wait()

    return gather_kernel(table, idx)


def _pre_body(x_ref, wzt_ref, bz_ref, wr_ref, bu_ref, wht_ref, bh_ref,
              xz_ref, xr_ref, xh_ref, h1_ref):
    x = x_ref[...]
    xz = x @ wzt_ref[...] + bz_ref[...]
    xr = x @ wr_ref[...] + bu_ref[...]
    xh = x @ wht_ref[...] + bh_ref[...]
    xz_ref[...] = xz
    xr_ref[...] = xr
    xh_ref[...] = xh
    h1 = jax.nn.sigmoid(xz) * jnp.tanh(xh)

    @pl.when(pl.program_id(0) == 0)
    def _zero_pad_row():
        row = lax.broadcasted_iota(jnp.int32, (x_ref.shape[0], 1), 0)
        h1_ref[...] = jnp.where(row == 0, 0.0, h1)

    @pl.when(pl.program_id(0) != 0)
    def _store():
        h1_ref[...] = h1


def _precompute(cur_x, W_z_w, W_z_b, W_r_w, U_r_b, W_h_w, W_h_b, blk):
    e_pad = cur_x.shape[0]
    grid = e_pad // blk

    def full(shape):
        return pl.BlockSpec(shape, lambda i: (0,) * len(shape))

    row_spec = pl.BlockSpec((blk, HIDDEN), lambda i: (i, 0))
    out = jax.ShapeDtypeStruct((e_pad, HIDDEN), jnp.float32)
    return pl.pallas_call(
        _pre_body,
        grid=(grid,),
        in_specs=[row_spec] + [full((HIDDEN, HIDDEN)), full((1, HIDDEN))] * 3,
        out_specs=[row_spec] * 4,
        out_shape=[out] * 4,
    )(cur_x, W_z_w[:HIDDEN], W_z_b.reshape(1, HIDDEN), W_r_w,
      U_r_b.reshape(1, HIDDEN), W_h_w[:HIDDEN], W_h_b.reshape(1, HIDDEN))


def _sigmoid(x):
    return 0.5 * jnp.tanh(0.5 * x) + 0.5


def _gru_body(xz_ref, xr_ref, xh_ref, hnei_ref, wzb_ref, ur_ref, whb_ref,
              *tail, zero_row):
    out_ref = tail[-1]
    blk = xz_ref.shape[0]
    h_nei = hnei_ref[...]
    sum_h = jnp.sum(h_nei, axis=0)
    z = _sigmoid(xz_ref[...] + sum_h @ wzb_ref[...])
    r2 = jnp.reshape(jnp.reshape(h_nei, (MAX_NEI * blk, HIDDEN)) @ ur_ref[...],
                     (MAX_NEI, blk, HIDDEN))
    r = _sigmoid(xr_ref[...][None] + r2)
    sum_gated = jnp.sum(r * h_nei, axis=0)
    pre_h = jnp.tanh(xh_ref[...] + sum_gated @ whb_ref[...])
    new_h = (1.0 - z) * sum_h + z * pre_h

    if zero_row:
        @pl.when(pl.program_id(0) == 0)
        def _zero_pad_row():
            row = lax.broadcasted_iota(jnp.int32, (blk, 1), 0)
            out_ref[...] = jnp.where(row == 0, 0.0, new_h)

        @pl.when(pl.program_id(0) != 0)
        def _store():
            out_ref[...] = new_h
    else:
        out_ref[...] = new_h


def _gru_step(xz, xr, xh, h_nei, wz_bot, ur_w, wh_bot, blk, zero_row,
              e_full, off_blocks=0, out_buf=None):
    n_e = xz.shape[0]
    blk = min(blk, n_e)
    grid = n_e // blk

    def full(shape):
        return pl.BlockSpec(shape, lambda i: (0,) * len(shape))

    row_spec = pl.BlockSpec((blk, HIDDEN), lambda i: (i, 0))
    in_specs = [
        row_spec, row_spec, row_spec,
        pl.BlockSpec((MAX_NEI, blk, HIDDEN), lambda i: (0, i, 0)),
        full((HIDDEN, HIDDEN)),
        full((HIDDEN, HIDDEN)),
        full((HIDDEN, HIDDEN)),
    ]
    args = [xz, xr, xh, h_nei, wz_bot, ur_w, wh_bot]
    aliases = {}
    if out_buf is not None:
        in_specs.append(pl.BlockSpec(memory_space=pl.ANY))
        args.append(out_buf)
        aliases = {7: 0}
    return pl.pallas_call(
        functools.partial(_gru_body, zero_row=zero_row),
        grid=(grid,),
        in_specs=in_specs,
        out_specs=pl.BlockSpec((blk, HIDDEN), lambda i: (i + off_blocks, 0)),
        out_shape=jax.ShapeDtypeStruct((e_full, HIDDEN), jnp.float32),
        input_output_aliases=aliases,
    )(*args)


def _root_body(x_ref, hnei_ref, w_ref, b_ref, out_ref):
    x = x_ref[...]
    sum_h = jnp.sum(hnei_ref[...], axis=0)
    w = w_ref[...]
    out = x @ w[:HIDDEN] + sum_h @ w[HIDDEN:] + b_ref[...]
    out_ref[...] = jnp.maximum(out, 0.0)


def _root_eval(root_x, root_h_nei, W_w, W_b):
    b = root_x.shape[0]
    return pl.pallas_call(
        _root_body,
        out_shape=jax.ShapeDtypeStruct((b, HIDDEN), jnp.float32),
    )(root_x, root_h_nei, W_w, W_b.reshape(1, HIDDEN))


def kernel(node_wid, fmess, mess_graph, root_idx, root_mess, emb,
           W_z_w, W_z_b, W_r_w, U_r_w, U_r_b, W_h_w, W_h_b, W_w, W_b):
    E = fmess.shape[0]
    BLK = 256
    e_pad = ((E + 14335) // 14336) * 14336

    pad_n = e_pad - E
    N = node_wid.shape[0]
    fm_pad = jnp.concatenate(
        [fmess.astype(jnp.int32), jnp.arange(pad_n, dtype=jnp.int32) % N])

    mg_pad2 = jnp.concatenate(
        [mess_graph.astype(jnp.int32),
         jnp.reshape(jnp.arange(pad_n * MAX_NEI, dtype=jnp.int32) % E,
                     (pad_n, MAX_NEI))])

    n_pad = ((N + 2047) // 2048) * 2048
    nw_pad = jnp.concatenate(
        [node_wid.astype(jnp.int32),
         jnp.arange(n_pad - N, dtype=jnp.int32) % emb.shape[0]])
    node_x = _sc_gather(emb, nw_pad, chunk=80)
    cur_x = _sc_gather(node_x, fm_pad, chunk=112)
    xz, xr, xh, h = _precompute(cur_x, W_z_w, W_z_b, W_r_w, U_r_b,
                                W_h_w, W_h_b, BLK)
    wz_bot = W_z_w[HIDDEN:]
    wh_bot = W_h_w[HIDDEN:]

    EA = e_pad // 4
    EB = e_pad - EA
    mgA = jnp.reshape(mg_pad2[:EA].T, (-1,))
    mgB = jnp.reshape(mg_pad2[EA:].T, (-1,))
    xzA, xzB = xz[:EA], xz[EA:]
    xrA, xrB = xr[:EA], xr[EA:]
    xhA, xhB = xh[:EA], xh[EA:]

    for _ in range(DEPTH - 1):
        gA = _sc_gather(h, mgA, chunk=112)
        gB = _sc_gather(h, mgB, chunk=112)
        hA = _gru_step(xzA, xrA, xhA,
                       jnp.reshape(gA, (MAX_NEI, EA, HIDDEN)),
                       wz_bot, U_r_w, wh_bot, BLK, True, e_pad)
        h = _gru_step(xzB, xrB, xhB,
                      jnp.reshape(gB, (MAX_NEI, EB, HIDDEN)),
                      wz_bot, U_r_w, wh_bot, BLK, False, e_pad,
                      off_blocks=EA // min(BLK, EB), out_buf=hA)

    B_r = root_idx.shape[0]
    ri_pad = jnp.concatenate(
        [root_idx.astype(jnp.int32),
         jnp.arange(B_r, dtype=jnp.int32) % N])
    root_x = _sc_gather(node_x, ri_pad, chunk=8)[:B_r]
    rm_t = jnp.reshape(root_mess.astype(jnp.int32).T, (-1,))
    root_h_nei_flat = _sc_gather(h, rm_t, chunk=32)
    root_h_nei = jnp.reshape(root_h_nei_flat,
                             (MAX_NEI, root_idx.shape[0], HIDDEN))
    root_vecs = _root_eval(root_x, root_h_nei, W_w, W_b)

    return (h[:E], root_vecs)


# speedup vs baseline: 4.7391x; 1.0424x over previous
import functools

import jax
import jax.numpy as jnp
from jax import lax
from jax.experimental import pallas as pl
from jax.experimental.pallas import tpu as pltpu
from jax.experimental.pallas import tpu_sc as plsc

HIDDEN = 128
DEPTH = 10
MAX_NEI = 8
NW = 32
NBUF = 4


@functools.partial(jax.jit, static_argnames=("chunk",))
def _sc_gather(table, idx, *, chunk):
    B = idx.shape[0]
    H = table.shape[1]
    dtype = table.dtype
    b_per_w = B // NW
    n_chunks = b_per_w // chunk
    n_outer = n_chunks // NBUF
    assert chunk <= 128 and chunk % 8 == 0, chunk
    assert B == NW * n_outer * NBUF * chunk, (B, chunk)
    mesh = plsc.VectorSubcoreMesh(core_axis_name="c", subcore_axis_name="s")

    @functools.partial(
        pl.kernel,
        out_type=jax.ShapeDtypeStruct((B, H), dtype),
        mesh=mesh,
        scratch_types=(
            [pltpu.VMEM((b_per_w,), jnp.int32)]
            + [pltpu.VMEM((chunk, H), dtype) for _ in range(NBUF)]
            + [pltpu.SemaphoreType.DMA for _ in range(2 * NBUF)]
        ),
    )
    def gather_kernel(table_hbm, idx_hbm, out_hbm, idx_all, *bufs):
        rows = bufs[:NBUF]
        gsem = bufs[NBUF:2 * NBUF]
        wsem = bufs[2 * NBUF:]
        wid = lax.axis_index("s") * 2 + lax.axis_index("c")
        base = wid * b_per_w
        pltpu.sync_copy(idx_hbm.at[pl.ds(base, b_per_w)], idx_all)

        def body(j, carry):
            for s in range(NBUF):
                @pl.when(j > 0)
                def _wait_wb(s=s):
                    pltpu.make_async_copy(
                        rows[s], out_hbm.at[pl.ds(base, chunk)], wsem[s]
                    ).wait()
                pltpu.async_copy(
                    table_hbm.at[idx_all.at[pl.ds((j * NBUF + s) * chunk,
                                                  chunk)]],
                    rows[s], gsem[s])
            for s in range(NBUF):
                pltpu.make_async_copy(
                    table_hbm.at[idx_all.at[pl.ds((j * NBUF + s) * chunk,
                                                  chunk)]],
                    rows[s], gsem[s]).wait()
                pltpu.async_copy(
                    rows[s],
                    out_hbm.at[pl.ds(base + (j * NBUF + s) * chunk, chunk)],
                    wsem[s])
            return carry

        lax.fori_loop(0, n_outer, body, 0)
        for s in range(NBUF):
            pltpu.make_async_copy(
                rows[s], out_hbm.at[pl.ds(base, chunk)], wsem[s]).wait()

    return gather_kernel(table, idx)


def _pre_body(x_ref, wzt_ref, bz_ref, wr_ref, bu_ref, wht_ref, bh_ref,
              xz_ref, xr_ref, xh_ref, h1_ref):
    x = x_ref[...]
    xz = x @ wzt_ref[...] + bz_ref[...]
    xr = x @ wr_ref[...] + bu_ref[...]
    xh = x @ wht_ref[...] + bh_ref[...]
    xz_ref[...] = xz
    xr_ref[...] = xr
    xh_ref[...] = xh
    h1 = jax.nn.sigmoid(xz) * jnp.tanh(xh)

    @pl.when(pl.program_id(0) == 0)
    def _zero_pad_row():
        row = lax.broadcasted_iota(jnp.int32, (x_ref.shape[0], 1), 0)
        h1_ref[...] = jnp.where(row == 0, 0.0, h1)

    @pl.when(pl.program_id(0) != 0)
    def _store():
        h1_ref[...] = h1


def _precompute(cur_x, W_z_w, W_z_b, W_r_w, U_r_b, W_h_w, W_h_b, blk):
    e_pad = cur_x.shape[0]
    grid = e_pad // blk

    def full(shape):
        return pl.BlockSpec(shape, lambda i: (0,) * len(shape))

    row_spec = pl.BlockSpec((blk, HIDDEN), lambda i: (i, 0))
    out = jax.ShapeDtypeStruct((e_pad, HIDDEN), jnp.float32)
    return pl.pallas_call(
        _pre_body,
        grid=(grid,),
        in_specs=[row_spec] + [full((HIDDEN, HIDDEN)), full((1, HIDDEN))] * 3,
        out_specs=[row_spec] * 4,
        out_shape=[out] * 4,
    )(cur_x, W_z_w[:HIDDEN], W_z_b.reshape(1, HIDDEN), W_r_w,
      U_r_b.reshape(1, HIDDEN), W_h_w[:HIDDEN], W_h_b.reshape(1, HIDDEN))


def _sigmoid(x):
    return 0.5 * jnp.tanh(0.5 * x) + 0.5


def _gru_body(xz_ref, xr_ref, xh_ref, hnei_ref, wzb_ref, ur_ref, whb_ref,
              *tail, zero_row):
    out_ref = tail[-1]
    blk = xz_ref.shape[0]
    h_nei = hnei_ref[...]
    sum_h = jnp.sum(h_nei, axis=0)
    z = _sigmoid(xz_ref[...] + sum_h @ wzb_ref[...])
    r2 = jnp.reshape(jnp.reshape(h_nei, (MAX_NEI * blk, HIDDEN)) @ ur_ref[...],
                     (MAX_NEI, blk, HIDDEN))
    r = _sigmoid(xr_ref[...][None] + r2)
    sum_gated = jnp.sum(r * h_nei, axis=0)
    pre_h = jnp.tanh(xh_ref[...] + sum_gated @ whb_ref[...])
    new_h = (1.0 - z) * sum_h + z * pre_h

    if zero_row:
        @pl.when(pl.program_id(0) == 0)
        def _zero_pad_row():
            row = lax.broadcasted_iota(jnp.int32, (blk, 1), 0)
            out_ref[...] = jnp.where(row == 0, 0.0, new_h)

        @pl.when(pl.program_id(0) != 0)
        def _store():
            out_ref[...] = new_h
    else:
        out_ref[...] = new_h


def _gru_step(xz, xr, xh, h_nei_seg, wz_bot, ur_w, wh_bot, blk, zero_row,
              off_blocks=0, out_buf=None):
    n_e = h_nei_seg.shape[1]
    e_full = xz.shape[0]
    blk = min(blk, n_e)
    grid = n_e // blk

    def full(shape):
        return pl.BlockSpec(shape, lambda i: (0,) * len(shape))

    row_spec = pl.BlockSpec((blk, HIDDEN), lambda i: (i + off_blocks, 0))
    in_specs = [
        row_spec, row_spec, row_spec,
        pl.BlockSpec((MAX_NEI, blk, HIDDEN), lambda i: (0, i, 0)),
        full((HIDDEN, HIDDEN)),
        full((HIDDEN, HIDDEN)),
        full((HIDDEN, HIDDEN)),
    ]
    args = [xz, xr, xh, h_nei_seg, wz_bot, ur_w, wh_bot]
    aliases = {}
    if out_buf is not None:
        in_specs.append(pl.BlockSpec(memory_space=pl.ANY))
        args.append(out_buf)
        aliases = {7: 0}
    return pl.pallas_call(
        functools.partial(_gru_body, zero_row=zero_row),
        grid=(grid,),
        in_specs=in_specs,
        out_specs=pl.BlockSpec((blk, HIDDEN), lambda i: (i + off_blocks, 0)),
        out_shape=jax.ShapeDtypeStruct((e_full, HIDDEN), jnp.float32),
        input_output_aliases=aliases,
    )(*args)


def _root_body(x_ref, hnei_ref, w_ref, b_ref, out_ref):
    x = x_ref[...]
    sum_h = jnp.sum(hnei_ref[...], axis=0)
    w = w_ref[...]
    out = x @ w[:HIDDEN] + sum_h @ w[HIDDEN:] + b_ref[...]
    out_ref[...] = jnp.maximum(out, 0.0)


def _root_eval(root_x, root_h_nei, W_w, W_b):
    b = root_x.shape[0]
    return pl.pallas_call(
        _root_body,
        out_shape=jax.ShapeDtypeStruct((b, HIDDEN), jnp.float32),
    )(root_x, root_h_nei, W_w, W_b.reshape(1, HIDDEN))


def kernel(node_wid, fmess, mess_graph, root_idx, root_mess, emb,
           W_z_w, W_z_b, W_r_w, U_r_w, U_r_b, W_h_w, W_h_b, W_w, W_b):
    E = fmess.shape[0]
    BLK = 256
    e_pad = ((E + 14335) // 14336) * 14336

    pad_n = e_pad - E
    N = node_wid.shape[0]
    fm_pad = jnp.concatenate(
        [fmess.astype(jnp.int32), jnp.arange(pad_n, dtype=jnp.int32) % N])

    mg_pad2 = jnp.concatenate(
        [mess_graph.astype(jnp.int32),
         jnp.reshape(jnp.arange(pad_n * MAX_NEI, dtype=jnp.int32) % E,
                     (pad_n, MAX_NEI))])

    n_pad = ((N + 2047) // 2048) * 2048
    nw_pad = jnp.concatenate(
        [node_wid.astype(jnp.int32),
         jnp.arange(n_pad - N, dtype=jnp.int32) % emb.shape[0]])
    node_x = _sc_gather(emb, nw_pad, chunk=80)
    cur_x = _sc_gather(node_x, fm_pad, chunk=112)
    xz, xr, xh, h = _precompute(cur_x, W_z_w, W_z_b, W_r_w, U_r_b,
                                W_h_w, W_h_b, BLK)
    wz_bot = W_z_w[HIDDEN:]
    wh_bot = W_h_w[HIDDEN:]

    n_blocks = e_pad // BLK
    seg_blocks = [n_blocks // 8, n_blocks // 4]
    seg_blocks.append(n_blocks - sum(seg_blocks))
    segs = []
    ob = 0
    for nb in seg_blocks:
        lo, hi = ob * BLK, (ob + nb) * BLK
        segs.append((ob, nb, hi - lo,
                     jnp.reshape(mg_pad2[lo:hi].T, (-1,))))
        ob += nb

    for _ in range(DEPTH - 1):
        gs = [_sc_gather(h, mg_s, chunk=112) for _, _, _, mg_s in segs]
        buf = None
        for s, (ob, nb, n_e, _) in enumerate(segs):
            buf = _gru_step(xz, xr, xh,
                            jnp.reshape(gs[s], (MAX_NEI, n_e, HIDDEN)),
                            wz_bot, U_r_w, wh_bot, BLK, s == 0,
                            off_blocks=ob, out_buf=buf)
        h = buf

    B_r = root_idx.shape[0]
    ri_pad = jnp.concatenate(
        [root_idx.astype(jnp.int32),
         jnp.arange(B_r, dtype=jnp.int32) % N])
    root_x = _sc_gather(node_x, ri_pad, chunk=8)[:B_r]
    rm_t = jnp.reshape(root_mess.astype(jnp.int32).T, (-1,))
    root_h_nei_flat = _sc_gather(h, rm_t, chunk=32)
    root_h_nei = jnp.reshape(root_h_nei_flat,
                             (MAX_NEI, root_idx.shape[0], HIDDEN))
    root_vecs = _root_eval(root_x, root_h_nei, W_w, W_b)

    return (h[:E], root_vecs)
